# baseline (speedup 1.0000x reference)
import functools

import jax
import jax.numpy as jnp
import numpy as np
from jax import lax
from jax.experimental import pallas as pl
from jax.experimental.pallas import tpu as pltpu
from jax.experimental.pallas import tpu_sc as plsc

_NC = 2
_NS = 16
_NW = _NC * _NS
_SUB = 128
_CHUNK = 8
_K = 2


def _perm():
    p = np.zeros(128, dtype=np.int32)
    for q in range(4):
        for k in range(16):
            p[32 * q + 2 * k] = 32 * q + k
            p[32 * q + 2 * k + 1] = 32 * q + 16 + k
    return p


def _ln(h, g, b, eps=1e-5):
    mu = jnp.mean(h, axis=-1, keepdims=True)
    var = jnp.mean((h - mu) ** 2, axis=-1, keepdims=True)
    return (h - mu) * lax.rsqrt(var + eps) * g + b


def _dense_tc(x, pe, W_in, b_in, W_att, ln1_g, ln1_b, W1, b1, W2, b2, ln2_g, ln2_b):
    n, d = x.shape
    dff = W1.shape[1]
    br = 1000

    def body(x_ref, pe_ref, wi, bi, wa, g1, be1, w1r, bv1, w2r, bv2, g2, be2, o_ref):
        h = jnp.dot(x_ref[...], wi[...], preferred_element_type=jnp.float32)
        h = h + bi[...] + pe_ref[...]
        a = jnp.dot(h, wa[...], preferred_element_type=jnp.float32)
        s1 = h + _ln(a, g1[...], be1[...])
        f = jnp.maximum(jnp.dot(s1, w1r[...], preferred_element_type=jnp.float32) + bv1[...], 0.0)
        f = jnp.dot(f, w2r[...], preferred_element_type=jnp.float32) + bv2[...]
        o_ref[...] = s1 + _ln(f, g2[...], be2[...])

    row_spec = pl.BlockSpec((br, d), lambda i: (i, 0))
    w_spec = lambda s: pl.BlockSpec(s, lambda i: (0, 0))
    return pl.pallas_call(
        body,
        grid=(n // br,),
        in_specs=[row_spec, row_spec, w_spec((d, d)), w_spec((1, d)),
                  w_spec((d, d)), w_spec((1, d)), w_spec((1, d)),
                  w_spec((d, dff)), w_spec((1, dff)), w_spec((dff, d)),
                  w_spec((1, d)), w_spec((1, d)), w_spec((1, d))],
        out_specs=row_spec,
        out_shape=jax.ShapeDtypeStruct((n, d), jnp.float32),
    )(x, pe, W_in, b_in, W_att, ln1_g, ln1_b, W1, b1, W2, b2, ln2_g, ln2_b)


def _norm_tc(deg0, deg1, src):
    n, d = src.shape
    br = 1000

    def body(d0, d1, s_ref, ss_ref, dinv_ref, selfw_ref):
        deg = d0[...] + d1[...] + 1.0
        pos = deg > 0
        dinv = jnp.where(pos, lax.rsqrt(jnp.where(pos, deg, 1.0)), 0.0)
        ss_ref[...] = dinv * s_ref[...]
        dinv_ref[...] = dinv
        selfw_ref[...] = dinv * dinv

    col_spec = pl.BlockSpec((br, 1), lambda i: (i, 0))
    row_spec = pl.BlockSpec((br, d), lambda i: (i, 0))
    return pl.pallas_call(
        body,
        grid=(n // br,),
        in_specs=[col_spec, col_spec, row_spec],
        out_specs=[row_spec, col_spec, col_spec],
        out_shape=[jax.ShapeDtypeStruct((n, d), jnp.float32),
                   jax.ShapeDtypeStruct((n, 1), jnp.float32),
                   jax.ShapeDtypeStruct((n, 1), jnp.float32)],
    )(deg0, deg1, src)


def _head_tc(p0, p1, src, dinv, selfw, W_out, b_out):
    n, d = src.shape
    dout = W_out.shape[1]
    br = 1000

    def body(p0r, p1r, s_ref, dv, sw, wo, bo, o_ref):
        pre = jnp.maximum(dv[...] * (p0r[...] + p1r[...]) + sw[...] * s_ref[...], 0.0)
        o_ref[...] = jnp.dot(pre, wo[...], preferred_element_type=jnp.float32) + bo[...]

    col_spec = pl.BlockSpec((br, 1), lambda i: (i, 0))
    row_spec = pl.BlockSpec((br, d), lambda i: (i, 0))
    return pl.pallas_call(
        body,
        grid=(n // br,),
        in_specs=[row_spec, row_spec, row_spec, col_spec, col_spec,
                  pl.BlockSpec((d, dout), lambda i: (0, 0)),
                  pl.BlockSpec((1, dout), lambda i: (0, 0))],
        out_specs=pl.BlockSpec((br, dout), lambda i: (i, 0)),
        out_shape=jax.ShapeDtypeStruct((n, dout), jnp.float32),
    )(p0, p1, src, dinv, selfw, W_out, b_out)


def _deg_sc(rowp, wp, zvec, npad, ept):
    mesh = plsc.VectorSubcoreMesh(core_axis_name="c", subcore_axis_name="s",
                                  num_cores=_NC, num_subcores=_NS)
    chunks = ept // (_CHUNK * _SUB)
    zb = npad // _NS

    @functools.partial(
        pl.kernel, mesh=mesh,
        out_type=jax.ShapeDtypeStruct((_NC * npad,), jnp.float32),
        scratch_types=[pltpu.VMEM((_CHUNK, _SUB), jnp.int32),
                       pltpu.VMEM((_CHUNK, _SUB), jnp.float32),
                       pltpu.VMEM((npad // _NS,), jnp.float32),
                       pltpu.VMEM_SHARED((npad,), jnp.float32)],
    )
    def k(row_hbm, w_hbm, z_hbm, out_hbm, idx_v, w_v, stage_v, deg_sh):
        cid = lax.axis_index("c")
        sid = lax.axis_index("s")
        wid = sid * _NC + cid
        zb0 = pl.multiple_of(sid * zb, 8)

        pltpu.sync_copy(z_hbm.at[pl.ds(zb0, zb)], stage_v)
        pltpu.sync_copy(stage_v, deg_sh.at[pl.ds(zb0, zb)])
        plsc.subcore_barrier()

        def body(ci, carry):
            rb = pl.multiple_of((wid * ept + ci * _CHUNK * _SUB) // _SUB, 8)
            pltpu.sync_copy(row_hbm.at[pl.ds(rb, _CHUNK)], idx_v)
            pltpu.sync_copy(w_hbm.at[pl.ds(rb, _CHUNK)], w_v)
            for j in range(_CHUNK):
                pltpu.sync_copy(w_v.at[j], deg_sh.at[idx_v.at[j]], add=True)
            return carry
        lax.fori_loop(0, chunks, body, 0)
        plsc.subcore_barrier()

        pltpu.sync_copy(deg_sh.at[pl.ds(zb0, zb)], stage_v)
        pltpu.sync_copy(stage_v,
                        out_hbm.at[pl.ds(pl.multiple_of(cid * npad + sid * zb, 8), zb)])

    return k(rowp, wp, zvec)


def _prop_sc(colp, rowp, wp, srcs_bf, zmat, npad, ept):
    d = srcs_bf.shape[1]
    mesh = plsc.VectorSubcoreMesh(core_axis_name="c", subcore_axis_name="s",
                                  num_cores=_NC, num_subcores=_NS)
    chunks = ept // (_CHUNK * _SUB)
    pairs = chunks // 2
    rows_per_tile = npad // _NS

    @functools.partial(
        pl.kernel, mesh=mesh,
        out_type=jax.ShapeDtypeStruct((_NC, npad, d), jnp.float32),
        scratch_types=[pltpu.VMEM((_CHUNK, _SUB), jnp.int32),
                       pltpu.VMEM((_CHUNK, _SUB), jnp.int32),
                       pltpu.VMEM((_CHUNK, _SUB), jnp.int32),
                       pltpu.VMEM((_CHUNK, _SUB), jnp.float32)]
                      + [pltpu.VMEM((_SUB, d), jnp.float32) for _ in range(_K)]
                      + [pltpu.VMEM_SHARED((npad, d), jnp.float32)]
                      + [pltpu.SemaphoreType.DMA for _ in range(5)],
    )
    def k(col_hbm, row_hbm, w_hbm, src_hbm, z_hbm, out_hbm,
          colA, colB, row_v, w_v, *rest):
        bufs = rest[:_K]
        acc_sh = rest[_K]
        sem_gs = rest[_K + 1:_K + 5]
        sem_s = rest[_K + 5]

        def gather4(idx_row, buf):
            for h in range(4):
                pltpu.async_copy(src_hbm.at[idx_row.at[pl.ds(32 * h, 32)]],
                                 buf.at[pl.ds(32 * h, 32)], sem_gs[h])

        def wait4(buf):
            for h in range(4):
                pltpu.make_async_copy(src_hbm.at[pl.ds(0, 32)],
                                      buf.at[pl.ds(32 * h, 32)],
                                      sem_gs[h]).wait()
        cid = lax.axis_index("c")
        sid = lax.axis_index("s")
        wid = sid * _NC + cid

        rt0 = pl.multiple_of(sid * rows_per_tile, 8)
        pltpu.sync_copy(z_hbm.at[pl.ds(rt0, rows_per_tile)],
                        acc_sh.at[pl.ds(rt0, rows_per_tile)])
        plsc.subcore_barrier()

        tile_rb = pl.multiple_of(wid * ept // _SUB, 8)

        def chunk_rb(ci):
            return pl.multiple_of(tile_rb + ci * _CHUNK, 8)

        pltpu.sync_copy(col_hbm.at[pl.ds(tile_rb, _CHUNK)], colA)
        for j in range(_K):
            gather4(colA.at[j], bufs[j])

        def do_chunk(ci, cur, nxt, maybe_last):
            if maybe_last:
                @pl.when(ci < chunks - 1)
                def _():
                    pltpu.sync_copy(col_hbm.at[pl.ds(chunk_rb(ci + 1), _CHUNK)], nxt)
            else:
                pltpu.sync_copy(col_hbm.at[pl.ds(chunk_rb(ci + 1), _CHUNK)], nxt)
            pltpu.sync_copy(row_hbm.at[pl.ds(chunk_rb(ci), _CHUNK)], row_v)
            pltpu.sync_copy(w_hbm.at[pl.ds(chunk_rb(ci), _CHUNK)], w_v)

            for j in range(_CHUNK):
                buf = bufs[j % _K]
                wait4(buf)

                def scale(eg, c2, buf=buf, j=j):
                    w16 = w_v[j, pl.ds(eg * 16, 16)]
                    for l in range(16):
                        wb = jnp.broadcast_to(w16[l], (16,))
                        r = eg * 16 + l
                        for q in range(d // 16):
                            sl = pl.ds(q * 16, 16)
                            buf[r, sl] = buf[r, sl] * wb
                    return c2
                lax.fori_loop(0, _SUB // 16, scale, 0)

                pltpu.async_copy(buf, acc_sh.at[row_v.at[j]], sem_s,
                                 add=True).wait()

                if j < _CHUNK - _K:
                    gather4(cur.at[j + _K], buf)
                elif maybe_last:
                    @pl.when(ci < chunks - 1)
                    def _(buf=buf, j=j):
                        gather4(nxt.at[j - (_CHUNK - _K)], buf)
                else:
                    gather4(nxt.at[j - (_CHUNK - _K)], buf)

        def body(m, carry):
            do_chunk(2 * m, colA, colB, False)
            do_chunk(2 * m + 1, colB, colA, True)
            return carry
        lax.fori_loop(0, pairs, body, 0)
        plsc.subcore_barrier()

        pltpu.sync_copy(acc_sh.at[pl.ds(rt0, rows_per_tile)],
                        out_hbm.at[cid, pl.ds(rt0, rows_per_tile)])

    return k(colp, rowp, wp, srcs_bf, zmat)


def kernel(x, edge_index, edge_attr, num_nodes, pe, W_in, b_in, W_att, ln1_g,
           ln1_b, W1, b1, W2, b2, ln2_g, ln2_b, W_out, b_out):
    n, d = x.shape[0], W_in.shape[1]
    e = edge_index.shape[1]
    ept = -(-e // (_NW * _CHUNK * _SUB)) * (_CHUNK * _SUB)
    epad = ept * _NW - e
    npad = -(-n // (8 * _NS)) * (8 * _NS)
    row = jnp.concatenate([edge_index[0], jnp.zeros((epad,), edge_index.dtype)])
    col = jnp.concatenate([edge_index[1], jnp.zeros((epad,), edge_index.dtype)])
    w = jnp.concatenate([edge_attr, jnp.zeros((epad,), edge_attr.dtype)])
    rowp = row.reshape(-1, _SUB)
    colp = col.reshape(-1, _SUB)
    wp = w.reshape(-1, _SUB)
    zvec = jnp.zeros((npad,), jnp.float32)
    zmat = jnp.zeros((npad, d), jnp.float32)
    vec2 = lambda v: v.reshape(1, -1)

    degp = _deg_sc(rowp, wp, zvec, npad, ept)
    src = _dense_tc(x, pe[:n], W_in, vec2(b_in), W_att, vec2(ln1_g), vec2(ln1_b),
                    W1, vec2(b1), W2, vec2(b2), vec2(ln2_g), vec2(ln2_b))
    ssp, dinv, selfw = _norm_tc(degp[:n].reshape(n, 1),
                                degp[npad:npad + n].reshape(n, 1), src)
    P = _prop_sc(colp, rowp, wp, ssp, zmat, npad, ept)
    return _head_tc(P[0, :n], P[1, :n], src, dinv, selfw, W_out, vec2(b_out))


# speedup vs baseline: 1.0174x; 1.0174x over previous
import functools

import jax
import jax.numpy as jnp
import numpy as np
from jax import lax
from jax.experimental import pallas as pl
from jax.experimental.pallas import tpu as pltpu
from jax.experimental.pallas import tpu_sc as plsc

_NC = 2
_NS = 16
_NW = _NC * _NS
_SUB = 128
_CHUNK = 8
_K = 2


def _perm():
    p = np.zeros(128, dtype=np.int32)
    for q in range(4):
        for k in range(16):
            p[32 * q + 2 * k] = 32 * q + k
            p[32 * q + 2 * k + 1] = 32 * q + 16 + k
    return p


def _ln(h, g, b, eps=1e-5):
    mu = jnp.mean(h, axis=-1, keepdims=True)
    var = jnp.mean((h - mu) ** 2, axis=-1, keepdims=True)
    return (h - mu) * lax.rsqrt(var + eps) * g + b


def _dense_tc(x, pe, W_in, b_in, W_att, ln1_g, ln1_b, W1, b1, W2, b2, ln2_g, ln2_b):
    n, d = x.shape
    dff = W1.shape[1]
    br = 1000

    def body(x_ref, pe_ref, wi, bi, wa, g1, be1, w1r, bv1, w2r, bv2, g2, be2, o_ref):
        h = jnp.dot(x_ref[...], wi[...], preferred_element_type=jnp.float32)
        h = h + bi[...] + pe_ref[...]
        a = jnp.dot(h, wa[...], preferred_element_type=jnp.float32)
        s1 = h + _ln(a, g1[...], be1[...])
        f = jnp.maximum(jnp.dot(s1, w1r[...], preferred_element_type=jnp.float32) + bv1[...], 0.0)
        f = jnp.dot(f, w2r[...], preferred_element_type=jnp.float32) + bv2[...]
        o_ref[...] = s1 + _ln(f, g2[...], be2[...])

    row_spec = pl.BlockSpec((br, d), lambda i: (i, 0))
    w_spec = lambda s: pl.BlockSpec(s, lambda i: (0, 0))
    return pl.pallas_call(
        body,
        grid=(n // br,),
        in_specs=[row_spec, row_spec, w_spec((d, d)), w_spec((1, d)),
                  w_spec((d, d)), w_spec((1, d)), w_spec((1, d)),
                  w_spec((d, dff)), w_spec((1, dff)), w_spec((dff, d)),
                  w_spec((1, d)), w_spec((1, d)), w_spec((1, d))],
        out_specs=row_spec,
        out_shape=jax.ShapeDtypeStruct((n, d), jnp.float32),
    )(x, pe, W_in, b_in, W_att, ln1_g, ln1_b, W1, b1, W2, b2, ln2_g, ln2_b)


def _norm_tc(deg0, deg1, src):
    n, d = src.shape
    br = 1000

    def body(d0, d1, s_ref, ss_ref, dinv_ref, selfw_ref):
        deg = d0[...] + d1[...] + 1.0
        pos = deg > 0
        dinv = jnp.where(pos, lax.rsqrt(jnp.where(pos, deg, 1.0)), 0.0)
        ss_ref[...] = dinv * s_ref[...]
        dinv_ref[...] = dinv
        selfw_ref[...] = dinv * dinv

    col_spec = pl.BlockSpec((br, 1), lambda i: (i, 0))
    row_spec = pl.BlockSpec((br, d), lambda i: (i, 0))
    return pl.pallas_call(
        body,
        grid=(n // br,),
        in_specs=[col_spec, col_spec, row_spec],
        out_specs=[row_spec, col_spec, col_spec],
        out_shape=[jax.ShapeDtypeStruct((n, d), jnp.float32),
                   jax.ShapeDtypeStruct((n, 1), jnp.float32),
                   jax.ShapeDtypeStruct((n, 1), jnp.float32)],
    )(deg0, deg1, src)


def _head_tc(p0, p1, src, dinv, selfw, W_out, b_out):
    n, d = src.shape
    dout = W_out.shape[1]
    br = 1000

    def body(p0r, p1r, s_ref, dv, sw, wo, bo, o_ref):
        pre = jnp.maximum(dv[...] * (p0r[...] + p1r[...]) + sw[...] * s_ref[...], 0.0)
        o_ref[...] = jnp.dot(pre, wo[...], preferred_element_type=jnp.float32) + bo[...]

    col_spec = pl.BlockSpec((br, 1), lambda i: (i, 0))
    row_spec = pl.BlockSpec((br, d), lambda i: (i, 0))
    return pl.pallas_call(
        body,
        grid=(n // br,),
        in_specs=[row_spec, row_spec, row_spec, col_spec, col_spec,
                  pl.BlockSpec((d, dout), lambda i: (0, 0)),
                  pl.BlockSpec((1, dout), lambda i: (0, 0))],
        out_specs=pl.BlockSpec((br, dout), lambda i: (i, 0)),
        out_shape=jax.ShapeDtypeStruct((n, dout), jnp.float32),
    )(p0, p1, src, dinv, selfw, W_out, b_out)


def _deg_sc(rowp, wp, zvec, npad, ept):
    mesh = plsc.VectorSubcoreMesh(core_axis_name="c", subcore_axis_name="s",
                                  num_cores=_NC, num_subcores=_NS)
    chunks = ept // (_CHUNK * _SUB)
    zb = npad // _NS

    @functools.partial(
        pl.kernel, mesh=mesh,
        out_type=jax.ShapeDtypeStruct((_NC * npad,), jnp.float32),
        scratch_types=[pltpu.VMEM((_CHUNK, _SUB), jnp.int32),
                       pltpu.VMEM((_CHUNK, _SUB), jnp.float32),
                       pltpu.VMEM((npad // _NS,), jnp.float32),
                       pltpu.VMEM_SHARED((npad,), jnp.float32)],
    )
    def k(row_hbm, w_hbm, z_hbm, out_hbm, idx_v, w_v, stage_v, deg_sh):
        cid = lax.axis_index("c")
        sid = lax.axis_index("s")
        wid = sid * _NC + cid
        zb0 = pl.multiple_of(sid * zb, 8)

        pltpu.sync_copy(z_hbm.at[pl.ds(zb0, zb)], stage_v)
        pltpu.sync_copy(stage_v, deg_sh.at[pl.ds(zb0, zb)])
        plsc.subcore_barrier()

        def body(ci, carry):
            rb = pl.multiple_of((wid * ept + ci * _CHUNK * _SUB) // _SUB, 8)
            pltpu.sync_copy(row_hbm.at[pl.ds(rb, _CHUNK)], idx_v)
            pltpu.sync_copy(w_hbm.at[pl.ds(rb, _CHUNK)], w_v)
            for j in range(_CHUNK):
                pltpu.sync_copy(w_v.at[j], deg_sh.at[idx_v.at[j]], add=True)
            return carry
        lax.fori_loop(0, chunks, body, 0)
        plsc.subcore_barrier()

        pltpu.sync_copy(deg_sh.at[pl.ds(zb0, zb)], stage_v)
        pltpu.sync_copy(stage_v,
                        out_hbm.at[pl.ds(pl.multiple_of(cid * npad + sid * zb, 8), zb)])

    return k(rowp, wp, zvec)


def _prop_sc(colp, rowp, wp, srcs_bf, zmat, npad, ept):
    d = srcs_bf.shape[1]
    mesh = plsc.VectorSubcoreMesh(core_axis_name="c", subcore_axis_name="s",
                                  num_cores=_NC, num_subcores=_NS)
    chunks = ept // (_CHUNK * _SUB)
    pairs = chunks // 2
    rows_per_tile = npad // _NS

    @functools.partial(
        pl.kernel, mesh=mesh,
        out_type=jax.ShapeDtypeStruct((_NC, npad, d), jnp.float32),
        scratch_types=[pltpu.VMEM((_CHUNK, _SUB), jnp.int32),
                       pltpu.VMEM((_CHUNK, _SUB), jnp.int32),
                       pltpu.VMEM((_CHUNK, _SUB), jnp.int32),
                       pltpu.VMEM((_CHUNK, _SUB), jnp.float32)]
                      + [pltpu.VMEM((_SUB, d), jnp.float32) for _ in range(_K)]
                      + [pltpu.VMEM_SHARED((npad, d), jnp.float32)]
                      + [pltpu.SemaphoreType.DMA for _ in range(5)],
    )
    def k(col_hbm, row_hbm, w_hbm, src_hbm, z_hbm, out_hbm,
          colA, colB, row_v, w_v, *rest):
        bufs = rest[:_K]
        acc_sh = rest[_K]
        sem_gs = rest[_K + 1:_K + 5]
        sem_s = rest[_K + 5]

        def gather4(idx_row, buf, p):
            pltpu.async_copy(src_hbm.at[idx_row], buf, sem_gs[p])

        def wait4(buf, p):
            pltpu.make_async_copy(src_hbm.at[pl.ds(0, _SUB)], buf,
                                  sem_gs[p]).wait()
        cid = lax.axis_index("c")
        sid = lax.axis_index("s")
        wid = sid * _NC + cid

        rt0 = pl.multiple_of(sid * rows_per_tile, 8)
        pltpu.sync_copy(z_hbm.at[pl.ds(rt0, rows_per_tile)],
                        acc_sh.at[pl.ds(rt0, rows_per_tile)])
        plsc.subcore_barrier()

        tile_rb = pl.multiple_of(wid * ept // _SUB, 8)

        def chunk_rb(ci):
            return pl.multiple_of(tile_rb + ci * _CHUNK, 8)

        pltpu.sync_copy(col_hbm.at[pl.ds(tile_rb, _CHUNK)], colA)
        for j in range(_K):
            gather4(colA.at[j], bufs[j], j % 2)

        def do_chunk(ci, cur, nxt, maybe_last):
            if maybe_last:
                @pl.when(ci < chunks - 1)
                def _():
                    pltpu.sync_copy(col_hbm.at[pl.ds(chunk_rb(ci + 1), _CHUNK)], nxt)
            else:
                pltpu.sync_copy(col_hbm.at[pl.ds(chunk_rb(ci + 1), _CHUNK)], nxt)
            pltpu.sync_copy(row_hbm.at[pl.ds(chunk_rb(ci), _CHUNK)], row_v)
            pltpu.sync_copy(w_hbm.at[pl.ds(chunk_rb(ci), _CHUNK)], w_v)

            for j in range(_CHUNK):
                buf = bufs[j % _K]
                wait4(buf, j % 2)

                def scale(eg, c2, buf=buf, j=j):
                    w16 = w_v[j, pl.ds(eg * 16, 16)]
                    for l in range(16):
                        wb = jnp.broadcast_to(w16[l], (16,))
                        r = eg * 16 + l
                        for q in range(d // 16):
                            sl = pl.ds(q * 16, 16)
                            buf[r, sl] = buf[r, sl] * wb
                    return c2
                lax.fori_loop(0, _SUB // 16, scale, 0)

                pltpu.async_copy(buf, acc_sh.at[row_v.at[j]], sem_s,
                                 add=True).wait()

                if j < _CHUNK - _K:
                    gather4(cur.at[j + _K], buf, j % 2)
                elif maybe_last:
                    @pl.when(ci < chunks - 1)
                    def _(buf=buf, j=j):
                        gather4(nxt.at[j - (_CHUNK - _K)], buf, j % 2)
                else:
                    gather4(nxt.at[j - (_CHUNK - _K)], buf, j % 2)

        def body(m, carry):
            do_chunk(2 * m, colA, colB, False)
            do_chunk(2 * m + 1, colB, colA, True)
            return carry
        lax.fori_loop(0, pairs, body, 0)
        plsc.subcore_barrier()

        pltpu.sync_copy(acc_sh.at[pl.ds(rt0, rows_per_tile)],
                        out_hbm.at[cid, pl.ds(rt0, rows_per_tile)])

    return k(colp, rowp, wp, srcs_bf, zmat)


def kernel(x, edge_index, edge_attr, num_nodes, pe, W_in, b_in, W_att, ln1_g,
           ln1_b, W1, b1, W2, b2, ln2_g, ln2_b, W_out, b_out):
    n, d = x.shape[0], W_in.shape[1]
    e = edge_index.shape[1]
    ept = -(-e // (_NW * _CHUNK * _SUB)) * (_CHUNK * _SUB)
    epad = ept * _NW - e
    npad = -(-n // (8 * _NS)) * (8 * _NS)
    row = jnp.concatenate([edge_index[0], jnp.zeros((epad,), edge_index.dtype)])
    col = jnp.concatenate([edge_index[1], jnp.zeros((epad,), edge_index.dtype)])
    w = jnp.concatenate([edge_attr, jnp.zeros((epad,), edge_attr.dtype)])
    rowp = row.reshape(-1, _SUB)
    colp = col.reshape(-1, _SUB)
    wp = w.reshape(-1, _SUB)
    zvec = jnp.zeros((npad,), jnp.float32)
    zmat = jnp.zeros((npad, d), jnp.float32)
    vec2 = lambda v: v.reshape(1, -1)

    degp = _deg_sc(rowp, wp, zvec, npad, ept)
    src = _dense_tc(x, pe[:n], W_in, vec2(b_in), W_att, vec2(ln1_g), vec2(ln1_b),
                    W1, vec2(b1), W2, vec2(b2), vec2(ln2_g), vec2(ln2_b))
    ssp, dinv, selfw = _norm_tc(degp[:n].reshape(n, 1),
                                degp[npad:npad + n].reshape(n, 1), src)
    P = _prop_sc(colp, rowp, wp, ssp, zmat, npad, ept)
    return _head_tc(P[0, :n], P[1, :n], src, dinv, selfw, W_out, vec2(b_out))
